# initial kernel scaffold (unmeasured)
import numpy as np
import jax
import jax.numpy as jnp
from jax import lax
from jax.experimental import pallas as pl
from jax.experimental.pallas import tpu as pltpu

N_DEV = 16
B, SQ, DM = 2, 256, 768
HQ_SHARD = 4
DH = 64
HD_SHARD = HQ_SHARD * DH
CH = SQ // N_DEV


def _tables():
    inv = 1.0 / (10000.0 ** (np.arange(0, DH, 2) / DH))
    pos = np.arange(SQ)[:, None] * inv[None, :]
    cos = np.repeat(np.cos(pos), 2, axis=-1)
    sin = np.repeat(np.sin(pos), 2, axis=-1)
    cos_t = np.tile(cos, (1, HQ_SHARD)).astype(np.float32)
    sin_t = np.tile(sin, (1, HQ_SHARD)).astype(np.float32)
    P = np.zeros((HD_SHARD, HD_SHARD), np.float32)
    for c in range(0, HD_SHARD, 2):
        P[c + 1, c] = -1.0
        P[c, c + 1] = 1.0
    return cos_t, sin_t, P


def kernel(x, Wq, Wk, Wv, Wo):
    cos_t, sin_t, P = _tables()
    f32 = jnp.float32

    def body(x_ref, wq_ref, wk_ref, wv_ref, wo_ref, cos_ref, sin_ref, p_ref,
             out_ref, partial_ref, reduced_ref, ctx_ref, rs_buf, ag_buf,
             rs_send, rs_recv, ag_send, ag_recv):
        my_d = lax.axis_index("i")
        cos = cos_ref[:, :]
        sin = sin_ref[:, :]
        pmat = p_ref[:, :]

        for b in range(B):
            xb = x_ref[b]
            q = jnp.dot(xb, wq_ref[:, :], preferred_element_type=f32)
            k = jnp.dot(xb, wk_ref[:, :], preferred_element_type=f32)
            v = jnp.dot(xb, wv_ref[:, :], preferred_element_type=f32)
            q = q * cos + jnp.dot(q, pmat, preferred_element_type=f32) * sin
            k = k * cos + jnp.dot(k, pmat, preferred_element_type=f32) * sin
            for h in range(HQ_SHARD):
                sl = slice(h * DH, (h + 1) * DH)
                qh, kh, vh = q[:, sl], k[:, sl], v[:, sl]
                s = lax.dot_general(qh, kh, (((1,), (1,)), ((), ())),
                                    preferred_element_type=f32) * 0.125
                m = jnp.max(s, axis=-1, keepdims=True)
                w = jnp.exp(s - m)
                w = w / jnp.sum(w, axis=-1, keepdims=True)
                ctx_ref[:, sl] = jnp.dot(w, vh, preferred_element_type=f32)
            partial_ref[b] = jnp.dot(ctx_ref[:, :], wo_ref[:, :],
                                     preferred_element_type=f32)

        rs_rdmas = []
        for k in range(1, N_DEV):
            t = lax.rem(my_d + k, N_DEV)
            rdma = pltpu.make_async_remote_copy(
                src_ref=partial_ref.at[:, pl.ds(t * CH, CH), :],
                dst_ref=rs_buf.at[k],
                send_sem=rs_send.at[k],
                recv_sem=rs_recv.at[k],
                device_id=(t,),
                device_id_type=pl.DeviceIdType.MESH,
            )
            rdma.start()
            rs_rdmas.append(rdma)

        acc = partial_ref[:, pl.ds(my_d * CH, CH), :]
        for k in range(1, N_DEV):
            rs_rdmas[k - 1].wait_recv()
            acc = acc + rs_buf[k]
        reduced_ref[:, :, :] = acc
        out_ref[:, pl.ds(my_d * CH, CH), :] = acc

        ag_rdmas = []
        for k in range(1, N_DEV):
            t = lax.rem(my_d + k, N_DEV)
            rdma = pltpu.make_async_remote_copy(
                src_ref=reduced_ref,
                dst_ref=ag_buf.at[k],
                send_sem=ag_send.at[k],
                recv_sem=ag_recv.at[k],
                device_id=(t,),
                device_id_type=pl.DeviceIdType.MESH,
            )
            rdma.start()
            ag_rdmas.append(rdma)

        for k in range(1, N_DEV):
            ag_rdmas[k - 1].wait_recv()
            c = lax.rem(my_d - k + N_DEV, N_DEV)
            out_ref[:, pl.ds(c * CH, CH), :] = ag_buf[k]

        for r in rs_rdmas:
            r.wait_send()
        for r in ag_rdmas:
            r.wait_send()

    return pl.pallas_call(
        body,
        out_shape=jax.ShapeDtypeStruct((B, SQ, DM), f32),
        in_specs=[pl.BlockSpec(memory_space=pltpu.VMEM)] * 8,
        out_specs=pl.BlockSpec(memory_space=pltpu.VMEM),
        scratch_shapes=[
            pltpu.VMEM((B, SQ, DM), f32),
            pltpu.VMEM((B, CH, DM), f32),
            pltpu.VMEM((SQ, HD_SHARD), f32),
            pltpu.VMEM((N_DEV, B, CH, DM), f32),
            pltpu.VMEM((N_DEV, B, CH, DM), f32),
            pltpu.SemaphoreType.DMA((N_DEV,)),
            pltpu.SemaphoreType.DMA((N_DEV,)),
            pltpu.SemaphoreType.DMA((N_DEV,)),
            pltpu.SemaphoreType.DMA((N_DEV,)),
        ],
        compiler_params=pltpu.CompilerParams(collective_id=0),
    )(x, Wq, Wk, Wv, Wo, jnp.asarray(cos_t), jnp.asarray(sin_t),
      jnp.asarray(P))


# baseline (device time: 56024 ns/iter reference)
import numpy as np
import jax
import jax.numpy as jnp
from jax import lax
from jax.experimental import pallas as pl
from jax.experimental.pallas import tpu as pltpu

N_DEV = 16
B, SQ, DM = 2, 256, 768
HQ_SHARD = 4
DH = 64
HD_SHARD = HQ_SHARD * DH
CH = SQ // N_DEV


def _tables():
    inv = 1.0 / (10000.0 ** (np.arange(0, DH, 2) / DH))
    pos = np.arange(SQ)[:, None] * inv[None, :]
    cos = np.repeat(np.cos(pos), 2, axis=-1)
    sin = np.repeat(np.sin(pos), 2, axis=-1)
    cos_t = np.tile(cos, (1, HQ_SHARD)).astype(np.float32)
    sin_t = np.tile(sin, (1, HQ_SHARD)).astype(np.float32)
    P = np.zeros((HD_SHARD, HD_SHARD), np.float32)
    for c in range(0, HD_SHARD, 2):
        P[c + 1, c] = -1.0
        P[c, c + 1] = 1.0
    return cos_t, sin_t, P


def kernel(x, Wq, Wk, Wv, Wo):
    cos_t, sin_t, P = _tables()
    f32 = jnp.float32

    def body(x_ref, wq_ref, wk_ref, wv_ref, wo_ref, cos_ref, sin_ref, p_ref,
             out_ref, partial_ref, reduced_ref, ctx_ref, rs_buf, ag_buf,
             rs_send, rs_recv, ag_send, ag_recv):
        my_d = lax.axis_index("i")
        cos = cos_ref[:, :]
        sin = sin_ref[:, :]
        pmat = p_ref[:, :]

        for b in range(B):
            xb = x_ref[b]
            q = jnp.dot(xb, wq_ref[:, :], preferred_element_type=f32)
            k = jnp.dot(xb, wk_ref[:, :], preferred_element_type=f32)
            v = jnp.dot(xb, wv_ref[:, :], preferred_element_type=f32)
            q = q * cos + jnp.dot(q, pmat, preferred_element_type=f32) * sin
            k = k * cos + jnp.dot(k, pmat, preferred_element_type=f32) * sin
            for h in range(HQ_SHARD):
                sl = slice(h * DH, (h + 1) * DH)
                qh, kh, vh = q[:, sl], k[:, sl], v[:, sl]
                s = lax.dot_general(qh, kh, (((1,), (1,)), ((), ())),
                                    preferred_element_type=f32) * 0.125
                m = jnp.max(s, axis=-1, keepdims=True)
                w = jnp.exp(s - m)
                w = w / jnp.sum(w, axis=-1, keepdims=True)
                ctx_ref[:, sl] = jnp.dot(w, vh, preferred_element_type=f32)
            partial_ref[b] = jnp.dot(ctx_ref[:, :], wo_ref[:, :],
                                     preferred_element_type=f32)

        rs_rdmas = []
        for k in range(1, N_DEV):
            t = lax.rem(my_d + k, N_DEV)
            rdma = pltpu.make_async_remote_copy(
                src_ref=partial_ref.at[:, pl.ds(t * CH, CH), :],
                dst_ref=rs_buf.at[k],
                send_sem=rs_send.at[k],
                recv_sem=rs_recv.at[k],
                device_id=(t,),
                device_id_type=pl.DeviceIdType.MESH,
            )
            rdma.start()
            rs_rdmas.append(rdma)

        acc = partial_ref[:, pl.ds(my_d * CH, CH), :]
        for k in range(1, N_DEV):
            rs_rdmas[k - 1].wait_recv()
            acc = acc + rs_buf[k]
        reduced_ref[:, :, :] = acc
        out_ref[:, pl.ds(my_d * CH, CH), :] = acc

        ag_rdmas = []
        for k in range(1, N_DEV):
            t = lax.rem(my_d + k, N_DEV)
            rdma = pltpu.make_async_remote_copy(
                src_ref=reduced_ref,
                dst_ref=ag_buf.at[k],
                send_sem=ag_send.at[k],
                recv_sem=ag_recv.at[k],
                device_id=(t,),
                device_id_type=pl.DeviceIdType.MESH,
            )
            rdma.start()
            ag_rdmas.append(rdma)

        for k in range(1, N_DEV):
            ag_rdmas[k - 1].wait_recv()
            c = lax.rem(my_d - k + N_DEV, N_DEV)
            out_ref[:, pl.ds(c * CH, CH), :] = ag_buf[k]

        for r in rs_rdmas:
            r.wait_send()
        for r in ag_rdmas:
            r.wait_send()

    return pl.pallas_call(
        body,
        out_shape=jax.ShapeDtypeStruct((B, SQ, DM), f32),
        in_specs=[pl.BlockSpec(memory_space=pltpu.VMEM)] * 8,
        out_specs=pl.BlockSpec(memory_space=pltpu.VMEM),
        scratch_shapes=[
            pltpu.VMEM((B, SQ, DM), f32),
            pltpu.VMEM((B, CH, DM), f32),
            pltpu.VMEM((SQ, HD_SHARD), f32),
            pltpu.VMEM((N_DEV, B, CH, DM), f32),
            pltpu.VMEM((N_DEV, B, CH, DM), f32),
            pltpu.SemaphoreType.DMA((N_DEV,)),
            pltpu.SemaphoreType.DMA((N_DEV,)),
            pltpu.SemaphoreType.DMA((N_DEV,)),
            pltpu.SemaphoreType.DMA((N_DEV,)),
        ],
    )(x, Wq, Wk, Wv, Wo, jnp.asarray(cos_t), jnp.asarray(sin_t),
      jnp.asarray(P))


# device time: 40670 ns/iter; 1.3775x vs baseline; 1.3775x over previous
import numpy as np
import jax
import jax.numpy as jnp
from jax import lax
from jax.experimental import pallas as pl
from jax.experimental.pallas import tpu as pltpu

N_DEV = 16
B, SQ, DM = 2, 256, 768
HQ_SHARD = 4
DH = 64
HD_SHARD = HQ_SHARD * DH
CH = SQ // N_DEV


def _tables():
    inv = 1.0 / (10000.0 ** (np.arange(0, DH, 2) / DH))
    pos = np.arange(SQ)[:, None] * inv[None, :]
    cos = np.repeat(np.cos(pos), 2, axis=-1)
    sin = np.repeat(np.sin(pos), 2, axis=-1)
    cos_t = np.tile(cos, (1, HQ_SHARD)).astype(np.float32)
    sin_t = np.tile(sin, (1, HQ_SHARD)).astype(np.float32)
    P = np.zeros((HD_SHARD, HD_SHARD), np.float32)
    for c in range(0, HD_SHARD, 2):
        P[c + 1, c] = -1.0
        P[c, c + 1] = 1.0
    return cos_t, sin_t, P


def kernel(x, Wq, Wk, Wv, Wo):
    cos_t, sin_t, P = _tables()
    f32 = jnp.float32
    bf16 = jnp.bfloat16

    def body(x_ref, wq_ref, wk_ref, wv_ref, wo_ref, cos_ref, sin_ref, p_ref,
             out_ref, partial_ref, reduced_ref, ctx_ref, acc_ref,
             rs_buf, ag_buf, rs_send, rs_recv, ag_send, ag_recv):
        my_d = lax.axis_index("i")
        cos = cos_ref[:, :]
        sin = sin_ref[:, :]
        pmat = p_ref[:, :].astype(bf16)
        wo16 = wo_ref[:, :].astype(bf16)

        wq16 = wq_ref[:, :].astype(bf16)
        wk16 = wk_ref[:, :].astype(bf16)
        wv16 = wv_ref[:, :].astype(bf16)
        for b in range(B):
            xb = x_ref[b].astype(bf16)
            q = jnp.dot(xb, wq16, preferred_element_type=f32)
            k = jnp.dot(xb, wk16, preferred_element_type=f32)
            v = jnp.dot(xb, wv16, preferred_element_type=f32)
            q = q * cos + jnp.dot(q.astype(bf16), pmat,
                                  preferred_element_type=f32) * sin
            k = k * cos + jnp.dot(k.astype(bf16), pmat,
                                  preferred_element_type=f32) * sin
            q16, k16, v16 = q.astype(bf16), k.astype(bf16), v.astype(bf16)
            for h in range(HQ_SHARD):
                sl = slice(h * DH, (h + 1) * DH)
                s = lax.dot_general(q16[:, sl], k16[:, sl],
                                    (((1,), (1,)), ((), ())),
                                    preferred_element_type=f32) * 0.125
                m = jnp.max(s, axis=-1, keepdims=True)
                w = jnp.exp(s - m)
                w = w / jnp.sum(w, axis=-1, keepdims=True)
                ctx_ref[b, :, sl] = jnp.dot(w.astype(bf16), v16[:, sl],
                                            preferred_element_type=f32
                                            ).astype(bf16)

        rs_rdmas = []
        for k in range(1, N_DEV):
            t = lax.rem(my_d + k, N_DEV)
            for b in range(B):
                pc = jnp.dot(ctx_ref[b, pl.ds(t * CH, CH), :], wo16,
                             preferred_element_type=f32)
                partial_ref[b, pl.ds(t * CH, CH), :] = pc.astype(bf16)
            rdma = pltpu.make_async_remote_copy(
                src_ref=partial_ref.at[:, pl.ds(t * CH, CH), :],
                dst_ref=rs_buf.at[k],
                send_sem=rs_send.at[k],
                recv_sem=rs_recv.at[k],
                device_id=(t,),
                device_id_type=pl.DeviceIdType.MESH,
            )
            rdma.start()
            rs_rdmas.append(rdma)

        for b in range(B):
            acc_ref[b] = jnp.dot(ctx_ref[b, pl.ds(my_d * CH, CH), :], wo16,
                                 preferred_element_type=f32)
        for k in range(1, N_DEV):
            rs_rdmas[k - 1].wait_recv()
            acc_ref[:, :, :] = acc_ref[:, :, :] + rs_buf[k].astype(f32)

        acc = acc_ref[:, :, :]
        reduced_ref[:, :, :] = acc.astype(bf16)
        out_ref[:, pl.ds(my_d * CH, CH), :] = acc

        ag_rdmas = []
        for k in range(1, N_DEV):
            t = lax.rem(my_d + k, N_DEV)
            rdma = pltpu.make_async_remote_copy(
                src_ref=reduced_ref,
                dst_ref=ag_buf.at[k],
                send_sem=ag_send.at[k],
                recv_sem=ag_recv.at[k],
                device_id=(t,),
                device_id_type=pl.DeviceIdType.MESH,
            )
            rdma.start()
            ag_rdmas.append(rdma)

        for k in range(1, N_DEV):
            ag_rdmas[k - 1].wait_recv()
            c = lax.rem(my_d - k + N_DEV, N_DEV)
            out_ref[:, pl.ds(c * CH, CH), :] = ag_buf[k].astype(f32)

        for r in rs_rdmas:
            r.wait_send()
        for r in ag_rdmas:
            r.wait_send()

    return pl.pallas_call(
        body,
        out_shape=jax.ShapeDtypeStruct((B, SQ, DM), f32),
        in_specs=[pl.BlockSpec(memory_space=pltpu.VMEM)] * 8,
        out_specs=pl.BlockSpec(memory_space=pltpu.VMEM),
        scratch_shapes=[
            pltpu.VMEM((B, SQ, DM), bf16),
            pltpu.VMEM((B, CH, DM), bf16),
            pltpu.VMEM((B, SQ, HD_SHARD), bf16),
            pltpu.VMEM((B, CH, DM), f32),
            pltpu.VMEM((N_DEV, B, CH, DM), bf16),
            pltpu.VMEM((N_DEV, B, CH, DM), bf16),
            pltpu.SemaphoreType.DMA((N_DEV,)),
            pltpu.SemaphoreType.DMA((N_DEV,)),
            pltpu.SemaphoreType.DMA((N_DEV,)),
            pltpu.SemaphoreType.DMA((N_DEV,)),
        ],
    )(x, Wq, Wk, Wv, Wo, jnp.asarray(cos_t), jnp.asarray(sin_t),
      jnp.asarray(P))
